# Initial kernel scaffold; baseline (speedup 1.0000x reference)
import jax
import jax.numpy as jnp
from jax.experimental import pallas as pl


def kernel(x, edge_index, W1, b1, W2, b2):
    raise NotImplementedError("write your pallas kernel here")


# speedup vs baseline: 18.9255x; 18.9255x over previous
import functools

import jax
import jax.numpy as jnp
from jax import lax
from jax.experimental import pallas as pl
from jax.experimental.pallas import tpu as pltpu
from jax.experimental.pallas import tpu_sc as plsc

N, E, D_IN, D_H, D_OUT = 10000, 320000, 128, 128, 64
NC, NS = 2, 16
NW = NC * NS
N_PAD = 10240
RPT = N_PAD // NS
C = 400
DEG_W = 16
DF = 64


def _mesh():
    return plsc.VectorSubcoreMesh(core_axis_name="c", subcore_axis_name="s")


_SC_PARAMS = pltpu.CompilerParams(use_tc_tiling_on_sc=False)


def _sc_degree(dst, ones_c, zeros16):
    @functools.partial(
        pl.kernel,
        out_type=jax.ShapeDtypeStruct((NC, N_PAD, 8, DEG_W), jnp.float32),
        mesh=_mesh(),
        scratch_types=[
            pltpu.VMEM((C,), jnp.int32),
            pltpu.VMEM((C, DEG_W), jnp.float32),
            pltpu.VMEM_SHARED((N_PAD, DEG_W), jnp.float32),
        ],
        compiler_params=_SC_PARAMS,
    )
    def k(dst_h, ones_h, zeros_h, out_h, idx_v, ones_v, acc):
        c = lax.axis_index("c")
        s = lax.axis_index("s")
        gwid = c * NS + s
        ept = E // NW
        r0 = s * RPT
        pltpu.sync_copy(zeros_h.at[pl.ds(r0, RPT)], acc.at[pl.ds(r0, RPT)])
        pltpu.sync_copy(ones_h, ones_v)
        plsc.subcore_barrier()

        def body(kk, carry):
            base = pl.multiple_of(gwid * ept + kk * C, 8)
            pltpu.sync_copy(dst_h.at[pl.ds(base, C)], idx_v)
            pltpu.sync_copy(ones_v, acc.at[idx_v], add=True)
            return carry

        lax.fori_loop(0, ept // C, body, 0)
        plsc.subcore_barrier()
        for j in range(8):
            pltpu.sync_copy(acc.at[pl.ds(r0, RPT)],
                            out_h.at[c, pl.ds(r0, RPT), j])

    return k(dst, ones_c, zeros16)


def _sc_aggregate(table, src, dst, zeros, feature_split):
    ept = (E // NS) if feature_split else (E // NW)

    @functools.partial(
        pl.kernel,
        out_type=jax.ShapeDtypeStruct((NC, N_PAD, DF), jnp.float32),
        mesh=_mesh(),
        scratch_types=[
            pltpu.VMEM((C,), jnp.int32),
            pltpu.VMEM((C,), jnp.int32),
            pltpu.VMEM((C, DF), jnp.float32),
            pltpu.VMEM_SHARED((N_PAD, DF), jnp.float32),
            pltpu.SemaphoreType.DMA,
        ],
        compiler_params=_SC_PARAMS,
    )
    def k(table_h, src_h, dst_h, zeros_h, out_h, src_v, dst_v, rows_v, acc,
          sem):
        c = lax.axis_index("c")
        s = lax.axis_index("s")
        gwid = s if feature_split else c * NS + s
        r0 = s * RPT
        pltpu.sync_copy(zeros_h.at[pl.ds(r0, RPT)], acc.at[pl.ds(r0, RPT)])
        plsc.subcore_barrier()

        def body(kk, carry):
            base = pl.multiple_of(gwid * ept + kk * C, 8)
            pltpu.sync_copy(src_h.at[pl.ds(base, C)], src_v)
            pltpu.sync_copy(dst_h.at[pl.ds(base, C)], dst_v)
            if feature_split:
                off = jnp.broadcast_to(c * N_PAD, (16,)).astype(jnp.int32)
                for j in range(C // 16):
                    sl = pl.ds(j * 16, 16)
                    src_v[sl] = src_v[sl] + off
            pltpu.async_copy(table_h.at[src_v], rows_v, sem).wait()
            pltpu.sync_copy(rows_v, acc.at[dst_v], add=True)
            return carry

        lax.fori_loop(0, ept // C, body, 0)
        plsc.subcore_barrier()
        pltpu.sync_copy(acc.at[pl.ds(r0, RPT)], out_h.at[c, pl.ds(r0, RPT)])

    return k(table, src, dst, zeros)


def _tc_layer1(xp, W1, deg128):
    BN = 1024
    grid = (N_PAD // BN,)

    def body(x_ref, w_ref, d_ref, h_ref, dinv_ref):
        d = d_ref[0] + d_ref[1]
        deg = jnp.sum(d, axis=1, keepdims=True) + 1.0
        dinv = jax.lax.rsqrt(deg)
        dinv128 = jnp.broadcast_to(dinv, (BN, 128))
        h = jnp.dot(x_ref[...], w_ref[...], preferred_element_type=jnp.float32)
        h = h * dinv128
        h_ref[0] = h[:, :DF]
        h_ref[1] = h[:, DF:]
        dinv_ref[...] = dinv128

    return pl.pallas_call(
        body,
        grid=grid,
        in_specs=[
            pl.BlockSpec((BN, D_IN), lambda i: (i, 0)),
            pl.BlockSpec((D_IN, D_H), lambda i: (0, 0)),
            pl.BlockSpec((NC, BN, 128), lambda i: (0, i, 0)),
        ],
        out_specs=[
            pl.BlockSpec((2, BN, DF), lambda i: (0, i, 0)),
            pl.BlockSpec((BN, 128), lambda i: (i, 0)),
        ],
        out_shape=[
            jax.ShapeDtypeStruct((2, N_PAD, DF), jnp.float32),
            jax.ShapeDtypeStruct((N_PAD, 128), jnp.float32),
        ],
    )(xp, W1, deg128)


def _tc_layer2(agg1, h1p, dinv128, b1, W2):
    BN = 1024
    grid = (N_PAD // BN,)

    def body(a_ref, h_ref, d_ref, b_ref, w_ref, out_ref):
        dinv = d_ref[...]
        full = jnp.concatenate([a_ref[0] + h_ref[0], a_ref[1] + h_ref[1]],
                               axis=1)
        z = full * dinv + b_ref[...]
        z = jnp.maximum(z, 0.0)
        h2 = jnp.dot(z, w_ref[...], preferred_element_type=jnp.float32)
        out_ref[...] = h2 * dinv[:, :D_OUT]

    return pl.pallas_call(
        body,
        grid=grid,
        in_specs=[
            pl.BlockSpec((NC, BN, DF), lambda i: (0, i, 0)),
            pl.BlockSpec((NC, BN, DF), lambda i: (0, i, 0)),
            pl.BlockSpec((BN, 128), lambda i: (i, 0)),
            pl.BlockSpec((1, D_H), lambda i: (0, 0)),
            pl.BlockSpec((D_H, D_OUT), lambda i: (0, 0)),
        ],
        out_specs=pl.BlockSpec((BN, D_OUT), lambda i: (i, 0)),
        out_shape=jax.ShapeDtypeStruct((N_PAD, D_OUT), jnp.float32),
    )(agg1, h1p, dinv128, b1, W2)


def _tc_head(agg2, h2p, dinv128, b2):
    BN = 1024
    grid = (N_PAD // BN,)

    def body(a_ref, h_ref, d_ref, b_ref, out_ref):
        dinv = d_ref[...][:, :D_OUT]
        y = (a_ref[0] + a_ref[1] + h_ref[...]) * dinv + b_ref[...]
        m = jnp.max(y, axis=1, keepdims=True)
        lse = jnp.log(jnp.sum(jnp.exp(y - m), axis=1, keepdims=True)) + m
        out_ref[...] = y - lse

    return pl.pallas_call(
        body,
        grid=grid,
        in_specs=[
            pl.BlockSpec((NC, BN, D_OUT), lambda i: (0, i, 0)),
            pl.BlockSpec((BN, D_OUT), lambda i: (i, 0)),
            pl.BlockSpec((BN, 128), lambda i: (i, 0)),
            pl.BlockSpec((1, D_OUT), lambda i: (0, 0)),
        ],
        out_specs=pl.BlockSpec((BN, D_OUT), lambda i: (i, 0)),
        out_shape=jax.ShapeDtypeStruct((N_PAD, D_OUT), jnp.float32),
    )(agg2, h2p, dinv128, b2)


def kernel(x, edge_index, W1, b1, W2, b2):
    src = edge_index[0]
    dst = edge_index[1]
    xp = jnp.pad(x, ((0, N_PAD - N), (0, 0)))
    ones_c = jnp.full((C, DEG_W), 1.0 / 128.0, jnp.float32)
    zeros16 = jnp.zeros((N_PAD, DEG_W), jnp.float32)
    zeros_f = jnp.zeros((N_PAD, DF), jnp.float32)

    degp = _sc_degree(dst, ones_c, zeros16)
    deg128 = degp.reshape(NC, N_PAD, 128)
    h1p, dinv128 = _tc_layer1(xp, W1, deg128)
    table1 = h1p.reshape(2 * N_PAD, DF)
    agg1 = _sc_aggregate(table1, src, dst, zeros_f, True)
    h2p = _tc_layer2(agg1, h1p, dinv128, b1.reshape(1, D_H), W2)
    agg2 = _sc_aggregate(h2p, src, dst, zeros_f, False)
    out = _tc_head(agg2, h2p, dinv128, b2.reshape(1, D_OUT))
    return out[:N]


# speedup vs baseline: 23.6178x; 1.2479x over previous
import functools

import jax
import jax.numpy as jnp
from jax import lax
from jax.experimental import pallas as pl
from jax.experimental.pallas import tpu as pltpu
from jax.experimental.pallas import tpu_sc as plsc

N, E, D_IN, D_H, D_OUT = 10000, 320000, 128, 128, 64
NC, NS = 2, 16
NW = NC * NS
N_PAD = 10240
RPT = N_PAD // NS
C = 400
DEG_W = 16
DF = 64


def _mesh():
    return plsc.VectorSubcoreMesh(core_axis_name="c", subcore_axis_name="s")


_SC_PARAMS = pltpu.CompilerParams(use_tc_tiling_on_sc=False)


def _sc_degree(dst, ones_c, zeros16):
    ept = E // NW

    @functools.partial(
        pl.kernel,
        out_type=jax.ShapeDtypeStruct((NC, N_PAD, 8, DEG_W), jnp.float32),
        mesh=_mesh(),
        scratch_types=[
            pltpu.VMEM((C,), jnp.int32),
            pltpu.VMEM((C,), jnp.int32),
            pltpu.VMEM((C, DEG_W), jnp.float32),
            pltpu.VMEM_SHARED((N_PAD, DEG_W), jnp.float32),
            pltpu.SemaphoreType.DMA,
        ],
        compiler_params=_SC_PARAMS,
    )
    def k(dst_h, ones_h, zeros_h, out_h, idx_a, idx_b, ones_v, acc, sem):
        c = lax.axis_index("c")
        s = lax.axis_index("s")
        gwid = c * NS + s
        r0 = s * RPT
        nk = ept // C
        pltpu.sync_copy(zeros_h.at[pl.ds(r0, RPT)], acc.at[pl.ds(r0, RPT)])
        pltpu.sync_copy(ones_h, ones_v)
        plsc.subcore_barrier()

        def load(kk, idx):
            base = pl.multiple_of(gwid * ept + kk * C, 8)
            pltpu.async_copy(dst_h.at[pl.ds(base, C)], idx, sem)

        def wait_load(kk, idx):
            base = pl.multiple_of(gwid * ept + kk * C, 8)
            pltpu.make_async_copy(dst_h.at[pl.ds(base, C)], idx, sem).wait()

        load(0, idx_a)

        def step(kk, cur, nxt):
            wait_load(kk, cur)

            @pl.when(kk + 1 < nk)
            def _():
                load(kk + 1, nxt)

            pltpu.sync_copy(ones_v, acc.at[cur], add=True)

        def body(kk, carry):
            @pl.when(kk % 2 == 0)
            def _():
                step(kk, idx_a, idx_b)

            @pl.when(kk % 2 == 1)
            def _():
                step(kk, idx_b, idx_a)

            return carry

        lax.fori_loop(0, nk, body, 0)
        plsc.subcore_barrier()
        for j in range(8):
            pltpu.sync_copy(acc.at[pl.ds(r0, RPT)],
                            out_h.at[c, pl.ds(r0, RPT), j])

    return k(dst, ones_c, zeros16)


def _sc_aggregate(table, src, dst, zeros, feature_split):
    ept = (E // NS) if feature_split else (E // NW)
    nk = ept // C

    @functools.partial(
        pl.kernel,
        out_type=jax.ShapeDtypeStruct((NC, N_PAD, DF), jnp.float32),
        mesh=_mesh(),
        scratch_types=[
            pltpu.VMEM((C,), jnp.int32),
            pltpu.VMEM((C,), jnp.int32),
            pltpu.VMEM((C,), jnp.int32),
            pltpu.VMEM((C, DF), jnp.float32),
            pltpu.VMEM((C, DF), jnp.float32),
            pltpu.VMEM_SHARED((N_PAD, DF), jnp.float32),
            pltpu.SemaphoreType.DMA,
        ],
        compiler_params=_SC_PARAMS,
    )
    def k(table_h, src_h, dst_h, zeros_h, out_h, src_a, src_b, dst_v,
          rows_a, rows_b, acc, sem_g):
        c = lax.axis_index("c")
        s = lax.axis_index("s")
        gwid = s if feature_split else c * NS + s
        r0 = s * RPT
        pltpu.sync_copy(zeros_h.at[pl.ds(r0, RPT)], acc.at[pl.ds(r0, RPT)])
        plsc.subcore_barrier()

        def load_gather(kk, idx, rows):
            base = pl.multiple_of(gwid * ept + kk * C, 8)
            pltpu.sync_copy(src_h.at[pl.ds(base, C)], idx)
            if feature_split:
                off = jnp.broadcast_to(c * N_PAD, (16,)).astype(jnp.int32)
                for j in range(C // 16):
                    sl = pl.ds(j * 16, 16)
                    idx[sl] = idx[sl] + off
            pltpu.async_copy(table_h.at[idx], rows, sem_g)

        def wait_gather(idx, rows):
            pltpu.make_async_copy(table_h.at[idx], rows, sem_g).wait()

        def scatter(kk, rows):
            base = pl.multiple_of(gwid * ept + kk * C, 8)
            pltpu.sync_copy(dst_h.at[pl.ds(base, C)], dst_v)
            pltpu.sync_copy(rows, acc.at[dst_v], add=True)

        load_gather(0, src_a, rows_a)

        def step(kk, idx_c, rows_c, idx_n, rows_n):
            @pl.when(kk + 1 < nk)
            def _():
                load_gather(kk + 1, idx_n, rows_n)

            wait_gather(idx_c, rows_c)
            scatter(kk, rows_c)

        def body(kk, carry):
            @pl.when(kk % 2 == 0)
            def _():
                step(kk, src_a, rows_a, src_b, rows_b)

            @pl.when(kk % 2 == 1)
            def _():
                step(kk, src_b, rows_b, src_a, rows_a)

            return carry

        lax.fori_loop(0, nk, body, 0)
        plsc.subcore_barrier()
        pltpu.sync_copy(acc.at[pl.ds(r0, RPT)], out_h.at[c, pl.ds(r0, RPT)])

    return k(table, src, dst, zeros)


def _tc_layer1(xp, W1, deg128):
    BN = 1024
    nb = N_PAD // BN
    grid = (nb, 2)

    def body(x_ref, w_ref, d_ref, t_ref, dinv_ref):
        d = d_ref[0] + d_ref[1]
        deg = jnp.sum(d, axis=1, keepdims=True) + 1.0
        dinv = jax.lax.rsqrt(deg)
        dinv128 = jnp.broadcast_to(dinv, (BN, 128))
        h = jnp.dot(x_ref[...], w_ref[0],
                    preferred_element_type=jnp.float32)
        t_ref[0] = h * dinv128[:, :DF]
        dinv_ref[...] = dinv128

    return pl.pallas_call(
        body,
        grid=grid,
        in_specs=[
            pl.BlockSpec((BN, D_IN), lambda i, j: (i, 0)),
            pl.BlockSpec((1, D_IN, DF), lambda i, j: (j, 0, 0)),
            pl.BlockSpec((NC, BN, 128), lambda i, j: (0, i, 0)),
        ],
        out_specs=[
            pl.BlockSpec((1, BN, DF), lambda i, j: (j, i, 0)),
            pl.BlockSpec((BN, 128), lambda i, j: (i, 0)),
        ],
        out_shape=[
            jax.ShapeDtypeStruct((2, N_PAD, DF), jnp.float32),
            jax.ShapeDtypeStruct((N_PAD, 128), jnp.float32),
        ],
    )(xp, W1, deg128)


def _tc_layer2(agg1, h1p, dinv128, b1, W2):
    BN = 1024
    grid = (N_PAD // BN,)

    def body(a_ref, h_ref, d_ref, b_ref, w_ref, out_ref):
        dinv = d_ref[...]
        full = jnp.concatenate([a_ref[0] + h_ref[0], a_ref[1] + h_ref[1]],
                               axis=1)
        z = full * dinv + b_ref[...]
        z = jnp.maximum(z, 0.0)
        h2 = jnp.dot(z, w_ref[...], preferred_element_type=jnp.float32)
        out_ref[...] = h2 * dinv[:, :D_OUT]

    return pl.pallas_call(
        body,
        grid=grid,
        in_specs=[
            pl.BlockSpec((NC, BN, DF), lambda i: (0, i, 0)),
            pl.BlockSpec((NC, BN, DF), lambda i: (0, i, 0)),
            pl.BlockSpec((BN, 128), lambda i: (i, 0)),
            pl.BlockSpec((1, D_H), lambda i: (0, 0)),
            pl.BlockSpec((D_H, D_OUT), lambda i: (0, 0)),
        ],
        out_specs=pl.BlockSpec((BN, D_OUT), lambda i: (i, 0)),
        out_shape=jax.ShapeDtypeStruct((N_PAD, D_OUT), jnp.float32),
    )(agg1, h1p, dinv128, b1, W2)


def _tc_head(agg2, h2p, dinv128, b2):
    BN = 1024
    grid = (N_PAD // BN,)

    def body(a_ref, h_ref, d_ref, b_ref, out_ref):
        dinv = d_ref[...][:, :D_OUT]
        y = (a_ref[0] + a_ref[1] + h_ref[...]) * dinv + b_ref[...]
        m = jnp.max(y, axis=1, keepdims=True)
        lse = jnp.log(jnp.sum(jnp.exp(y - m), axis=1, keepdims=True)) + m
        out_ref[...] = y - lse

    return pl.pallas_call(
        body,
        grid=grid,
        in_specs=[
            pl.BlockSpec((NC, BN, D_OUT), lambda i: (0, i, 0)),
            pl.BlockSpec((BN, D_OUT), lambda i: (i, 0)),
            pl.BlockSpec((BN, 128), lambda i: (i, 0)),
            pl.BlockSpec((1, D_OUT), lambda i: (0, 0)),
        ],
        out_specs=pl.BlockSpec((BN, D_OUT), lambda i: (i, 0)),
        out_shape=jax.ShapeDtypeStruct((N_PAD, D_OUT), jnp.float32),
    )(agg2, h2p, dinv128, b2)


def kernel(x, edge_index, W1, b1, W2, b2):
    src = edge_index[0]
    dst = edge_index[1]
    xp = jnp.pad(x, ((0, N_PAD - N), (0, 0)))
    ones_c = jnp.full((C, DEG_W), 1.0 / 128.0, jnp.float32)
    zeros16 = jnp.zeros((N_PAD, DEG_W), jnp.float32)
    zeros_f = jnp.zeros((N_PAD, DF), jnp.float32)

    W1h = W1.reshape(D_IN, 2, DF).transpose(1, 0, 2)
    degp = _sc_degree(dst, ones_c, zeros16)
    deg128 = degp.reshape(NC, N_PAD, 128)
    h1p, dinv128 = _tc_layer1(xp, W1h, deg128)
    table1 = h1p.reshape(2 * N_PAD, DF)
    agg1 = _sc_aggregate(table1, src, dst, zeros_f, True)
    h2p = _tc_layer2(agg1, h1p, dinv128, b1.reshape(1, D_H), W2)
    agg2 = _sc_aggregate(h2p, src, dst, zeros_f, False)
    out = _tc_head(agg2, h2p, dinv128, b2.reshape(1, D_OUT))
    return out[:N]


# speedup vs baseline: 30.2373x; 1.2803x over previous
import functools

import jax
import jax.numpy as jnp
from jax import lax
from jax.experimental import pallas as pl
from jax.experimental.pallas import tpu as pltpu
from jax.experimental.pallas import tpu_sc as plsc

N, E, D_IN, D_H, D_OUT = 10000, 320000, 128, 128, 64
NC, NS = 2, 16
NW = NC * NS
N_PAD = 10240
RPT = N_PAD // NS
C = 400
DEG_W = 16
DF = 64


def _mesh():
    return plsc.VectorSubcoreMesh(core_axis_name="c", subcore_axis_name="s")


_SC_PARAMS = pltpu.CompilerParams(use_tc_tiling_on_sc=False)


def _sc_degree(dst, ones_c, zeros16):
    ept = E // NW

    @functools.partial(
        pl.kernel,
        out_type=jax.ShapeDtypeStruct((NC, N_PAD, DEG_W), jnp.float32),
        mesh=_mesh(),
        scratch_types=[
            pltpu.VMEM((C,), jnp.int32),
            pltpu.VMEM((C,), jnp.int32),
            pltpu.VMEM((C, DEG_W), jnp.float32),
            pltpu.VMEM_SHARED((N_PAD, DEG_W), jnp.float32),
            pltpu.SemaphoreType.DMA,
        ],
        compiler_params=_SC_PARAMS,
    )
    def k(dst_h, ones_h, zeros_h, out_h, idx_a, idx_b, ones_v, acc, sem):
        c = lax.axis_index("c")
        s = lax.axis_index("s")
        gwid = c * NS + s
        r0 = s * RPT
        nk = ept // C
        pltpu.sync_copy(zeros_h.at[pl.ds(r0, RPT)], acc.at[pl.ds(r0, RPT)])
        pltpu.sync_copy(ones_h, ones_v)
        plsc.subcore_barrier()

        def load(kk, idx):
            base = pl.multiple_of(gwid * ept + kk * C, 8)
            pltpu.async_copy(dst_h.at[pl.ds(base, C)], idx, sem)

        def wait_load(kk, idx):
            base = pl.multiple_of(gwid * ept + kk * C, 8)
            pltpu.make_async_copy(dst_h.at[pl.ds(base, C)], idx, sem).wait()

        load(0, idx_a)

        def step(kk, cur, nxt):
            wait_load(kk, cur)

            @pl.when(kk + 1 < nk)
            def _():
                load(kk + 1, nxt)

            pltpu.sync_copy(ones_v, acc.at[cur], add=True)

        def body(kk, carry):
            @pl.when(kk % 2 == 0)
            def _():
                step(kk, idx_a, idx_b)

            @pl.when(kk % 2 == 1)
            def _():
                step(kk, idx_b, idx_a)

            return carry

        lax.fori_loop(0, nk, body, 0)
        plsc.subcore_barrier()
        pltpu.sync_copy(acc.at[pl.ds(r0, RPT)], out_h.at[c, pl.ds(r0, RPT)])

    return k(dst, ones_c, zeros16)


def _sc_aggregate(table, src, dst, zeros, feature_split):
    ept = (E // NS) if feature_split else (E // NW)
    nk = ept // C

    @functools.partial(
        pl.kernel,
        out_type=jax.ShapeDtypeStruct((NC, N_PAD, DF), jnp.float32),
        mesh=_mesh(),
        scratch_types=[
            pltpu.VMEM((C,), jnp.int32),
            pltpu.VMEM((C,), jnp.int32),
            pltpu.VMEM((C,), jnp.int32),
            pltpu.VMEM((C, DF), jnp.float32),
            pltpu.VMEM((C, DF), jnp.float32),
            pltpu.VMEM_SHARED((N_PAD, DF), jnp.float32),
            pltpu.SemaphoreType.DMA,
        ],
        compiler_params=_SC_PARAMS,
    )
    def k(table_h, src_h, dst_h, zeros_h, out_h, src_a, src_b, dst_v,
          rows_a, rows_b, acc, sem_g):
        c = lax.axis_index("c")
        s = lax.axis_index("s")
        gwid = s if feature_split else c * NS + s
        r0 = s * RPT
        pltpu.sync_copy(zeros_h.at[pl.ds(r0, RPT)], acc.at[pl.ds(r0, RPT)])
        plsc.subcore_barrier()

        def load_gather(kk, idx, rows):
            base = pl.multiple_of(gwid * ept + kk * C, 8)
            pltpu.sync_copy(src_h.at[pl.ds(base, C)], idx)
            if feature_split:
                off = jnp.broadcast_to(c * N_PAD, (16,)).astype(jnp.int32)
                for j in range(C // 16):
                    sl = pl.ds(j * 16, 16)
                    idx[sl] = idx[sl] + off
            pltpu.async_copy(table_h.at[idx], rows, sem_g)

        def wait_gather(idx, rows):
            pltpu.make_async_copy(table_h.at[idx], rows, sem_g).wait()

        def scatter(kk, rows):
            base = pl.multiple_of(gwid * ept + kk * C, 8)
            pltpu.sync_copy(dst_h.at[pl.ds(base, C)], dst_v)
            pltpu.sync_copy(rows, acc.at[dst_v], add=True)

        load_gather(0, src_a, rows_a)

        def step(kk, idx_c, rows_c, idx_n, rows_n):
            @pl.when(kk + 1 < nk)
            def _():
                load_gather(kk + 1, idx_n, rows_n)

            wait_gather(idx_c, rows_c)
            scatter(kk, rows_c)

        def body(kk, carry):
            @pl.when(kk % 2 == 0)
            def _():
                step(kk, src_a, rows_a, src_b, rows_b)

            @pl.when(kk % 2 == 1)
            def _():
                step(kk, src_b, rows_b, src_a, rows_a)

            return carry

        lax.fori_loop(0, nk, body, 0)
        plsc.subcore_barrier()
        pltpu.sync_copy(acc.at[pl.ds(r0, RPT)], out_h.at[c, pl.ds(r0, RPT)])

    return k(table, src, dst, zeros)


def _tc_layer1(xp, W1, deg16):
    BN = 1024
    nb = N_PAD // BN
    grid = (nb, 2)

    def body(x_ref, w_ref, d_ref, t_ref, dinv_ref):
        d = d_ref[0] + d_ref[1]
        deg = jnp.sum(d, axis=1, keepdims=True) + 1.0
        dinv = jax.lax.rsqrt(deg)
        h = jnp.dot(x_ref[...], w_ref[0],
                    preferred_element_type=jnp.float32)
        t_ref[0] = h * jnp.broadcast_to(dinv, (BN, DF))
        dinv_ref[...] = jnp.broadcast_to(dinv, (BN, DEG_W))

    return pl.pallas_call(
        body,
        grid=grid,
        in_specs=[
            pl.BlockSpec((BN, D_IN), lambda i, j: (i, 0)),
            pl.BlockSpec((1, D_IN, DF), lambda i, j: (j, 0, 0)),
            pl.BlockSpec((NC, BN, DEG_W), lambda i, j: (0, i, 0)),
        ],
        out_specs=[
            pl.BlockSpec((1, BN, DF), lambda i, j: (j, i, 0)),
            pl.BlockSpec((BN, DEG_W), lambda i, j: (i, 0)),
        ],
        out_shape=[
            jax.ShapeDtypeStruct((2, N_PAD, DF), jnp.float32),
            jax.ShapeDtypeStruct((N_PAD, DEG_W), jnp.float32),
        ],
    )(xp, W1, deg16)


def _tc_layer2(agg1, h1p, dinv16, b1, W2):
    BN = 1024
    grid = (N_PAD // BN,)

    def body(a_ref, h_ref, d_ref, b_ref, w_ref, out_ref):
        dinv = jnp.broadcast_to(d_ref[...][:, :1], (BN, D_H))
        full = jnp.concatenate([a_ref[0] + h_ref[0], a_ref[1] + h_ref[1]],
                               axis=1)
        z = full * dinv + b_ref[...]
        z = jnp.maximum(z, 0.0)
        h2 = jnp.dot(z, w_ref[...], preferred_element_type=jnp.float32)
        out_ref[...] = h2 * dinv[:, :D_OUT]

    return pl.pallas_call(
        body,
        grid=grid,
        in_specs=[
            pl.BlockSpec((NC, BN, DF), lambda i: (0, i, 0)),
            pl.BlockSpec((NC, BN, DF), lambda i: (0, i, 0)),
            pl.BlockSpec((BN, DEG_W), lambda i: (i, 0)),
            pl.BlockSpec((1, D_H), lambda i: (0, 0)),
            pl.BlockSpec((D_H, D_OUT), lambda i: (0, 0)),
        ],
        out_specs=pl.BlockSpec((BN, D_OUT), lambda i: (i, 0)),
        out_shape=jax.ShapeDtypeStruct((N_PAD, D_OUT), jnp.float32),
    )(agg1, h1p, dinv16, b1, W2)


def _tc_head(agg2, h2p, dinv16, b2):
    BN = 1024
    grid = (N_PAD // BN,)

    def body(a_ref, h_ref, d_ref, b_ref, out_ref):
        dinv = jnp.broadcast_to(d_ref[...][:, :1], (BN, D_OUT))
        y = (a_ref[0] + a_ref[1] + h_ref[...]) * dinv + b_ref[...]
        m = jnp.max(y, axis=1, keepdims=True)
        lse = jnp.log(jnp.sum(jnp.exp(y - m), axis=1, keepdims=True)) + m
        out_ref[...] = y - lse

    return pl.pallas_call(
        body,
        grid=grid,
        in_specs=[
            pl.BlockSpec((NC, BN, D_OUT), lambda i: (0, i, 0)),
            pl.BlockSpec((BN, D_OUT), lambda i: (i, 0)),
            pl.BlockSpec((BN, DEG_W), lambda i: (i, 0)),
            pl.BlockSpec((1, D_OUT), lambda i: (0, 0)),
        ],
        out_specs=pl.BlockSpec((BN, D_OUT), lambda i: (i, 0)),
        out_shape=jax.ShapeDtypeStruct((N_PAD, D_OUT), jnp.float32),
    )(agg2, h2p, dinv16, b2)


def kernel(x, edge_index, W1, b1, W2, b2):
    src = edge_index[0]
    dst = edge_index[1]
    xp = jnp.pad(x, ((0, N_PAD - N), (0, 0)))
    ones_c = jnp.full((C, DEG_W), 1.0 / DEG_W, jnp.float32)
    zeros16 = jnp.zeros((N_PAD, DEG_W), jnp.float32)
    zeros_f = jnp.zeros((N_PAD, DF), jnp.float32)

    W1h = W1.reshape(D_IN, 2, DF).transpose(1, 0, 2)
    deg16 = _sc_degree(dst, ones_c, zeros16)
    h1p, dinv16 = _tc_layer1(xp, W1h, deg16)
    table1 = h1p.reshape(2 * N_PAD, DF)
    agg1 = _sc_aggregate(table1, src, dst, zeros_f, True)
    h2p = _tc_layer2(agg1, h1p, dinv16, b1.reshape(1, D_H), W2)
    agg2 = _sc_aggregate(h2p, src, dst, zeros_f, False)
    out = _tc_head(agg2, h2p, dinv16, b2.reshape(1, D_OUT))
    return out[:N]
